# baseline (speedup 1.0000x reference)
import functools

import numpy as np
import jax
import jax.numpy as jnp
from jax import lax
from jax.experimental import pallas as pl
from jax.experimental.pallas import tpu as pltpu
from jax.experimental.pallas import tpu_sc as plsc

_J = 8
_L = 8
_L1 = 4

_C = 1792
_H = 224
_W = 224
_S = _H * _W


def _ratio_index_lists():
    par1, par2, per1, per2, q1, q2 = [], [], [], [], [], []
    for j1 in range(_J):
        for j2 in range(j1 + 1, _J):
            for l2 in range(_L):
                ci2 = (_L1 * _L * (_J - j1 - 1) + l2 + _L * (j2 - j1 - 1)
                       + _L ** 2 * (j1 * (_J - 1) - j1 * (j1 - 1) // 2))
                ci1 = _L1 + j1 * _L
                if l2 == _L1:
                    par1.append(ci1); par2.append(ci2)
                if l2 == _L1 + _L / 2 or l2 == _L1 - _L / 2:
                    per1.append(ci1); per2.append(ci2)
                if l2 == _L1 + _L // 4 or l2 == _L1 - _L // 4:
                    q1.append(ci1); q2.append(ci2)
    return (np.array(par1, np.int32), np.array(par2, np.int32),
            np.array(per1, np.int32), np.array(per2, np.int32),
            np.array(q1, np.int32), np.array(q2, np.int32))


_P1, _P2, _R1, _R2, _Q1, _Q2 = _ratio_index_lists()
_N_PAR = len(_P1)
_N_PER = len(_R1)
_N_QUART = len(_Q1)

_DEN = np.array(sorted(set(_P1) | set(_R1) | set(_Q1)), np.int32)
_NDEN = len(_DEN)
_DEN_POS = {int(c): i for i, c in enumerate(_DEN)}
_DPAD = 8 - _NDEN

_CH0 = np.concatenate([_P2, _R2, _DEN, np.zeros(_DPAD, np.int32)])
_CH1 = np.concatenate([_Q2, _DEN, np.zeros(_DPAD, np.int32)])
assert len(_CH0) == 64 and len(_CH1) == 64

_NUM0 = np.concatenate([np.arange(56, dtype=np.int32),
                        np.zeros(8, np.int32)])
_DENP0 = np.concatenate([
    np.array([56 + _DEN_POS[int(c)] for c in _P1], np.int32),
    np.array([56 + _DEN_POS[int(c)] for c in _R1], np.int32),
    np.zeros(8, np.int32)])
_NUM1 = np.concatenate([np.arange(56, dtype=np.int32),
                        np.zeros(8, np.int32)])
_DENP1 = np.concatenate([
    np.array([56 + _DEN_POS[int(c)] for c in _Q1], np.int32),
    np.zeros(8, np.int32)])

_CHAN = np.zeros((2, 16, 16), np.int32)
_CHAN[0, :, :4] = _CH0.reshape(16, 4)
_CHAN[1, :, :4] = _CH1.reshape(16, 4)
_NUMP = np.stack([_NUM0, _NUM1])
_DENP = np.stack([_DENP0, _DENP1])

_CPT = 4
_NCK = 4
_CHUNK = _S // _NCK
_RING = 4
_TPT = _CPT * _NCK


@functools.lru_cache(maxsize=1)
def _make_s2_kernel():
    mesh = plsc.VectorSubcoreMesh(core_axis_name="c", subcore_axis_name="s")

    @functools.partial(
        pl.kernel,
        mesh=mesh,
        compiler_params=pltpu.CompilerParams(needs_layout_passes=False),
        out_type=(jax.ShapeDtypeStruct((128,), jnp.float32),
                  jax.ShapeDtypeStruct((2, 16, 16), jnp.float32)),
        scratch_types=(
            [pltpu.VMEM((_CHUNK,), jnp.float32) for _ in range(_RING)]
            + [pltpu.SemaphoreType.DMA for _ in range(_RING)]
            + [
                pltpu.VMEM((16, 16), jnp.int32),
                pltpu.VMEM((64,), jnp.int32),
                pltpu.VMEM((64,), jnp.int32),
                pltpu.VMEM((16,), jnp.float32),
                pltpu.MemorySpace.VMEM_SHARED((16, 16), jnp.float32),
                pltpu.VMEM((16, 16), jnp.float32),
                pltpu.VMEM((64,), jnp.float32),
            ]
        ),
    )
    def _s2_kernel(x_hbm, chan_hbm, nump_hbm, denp_hbm, out_hbm, pub_hbm,
                   b0, b1, b2, b3, s0, s1, s2, s3,
                   chan_v, nump_v, denp_v, means_v,
                   shared, stage_v, ratio_v):
        bufs = (b0, b1, b2, b3)
        sems = (s0, s1, s2, s3)
        cid = lax.axis_index("c")
        sid = lax.axis_index("s")

        pltpu.sync_copy(chan_hbm.at[cid], chan_v)
        my_chans = chan_v[sid]

        def _copy(t, k):
            c_local = t // _NCK
            j = t % _NCK
            ch = my_chans[c_local]
            return pltpu.async_copy(
                x_hbm.at[ch, pl.ds(j * _CHUNK, _CHUNK)], bufs[k], sems[k])

        for k in range(_RING):
            _copy(k, k)

        lane = lax.iota(jnp.int32, 16)
        means_vec = jnp.zeros((16,), jnp.float32)
        for c_local in range(_CPT):
            acc = jnp.zeros((16,), jnp.float32)
            for j in range(_NCK):
                t = c_local * _NCK + j
                k = t % _RING
                pltpu.make_async_copy(
                    x_hbm.at[0, pl.ds(0, _CHUNK)], bufs[k], sems[k]).wait()

                def _inner(i, a, _k=k):
                    return a + bufs[_k][pl.ds(i * 16, 16)]

                acc = acc + lax.fori_loop(
                    0, _CHUNK // 16, _inner, jnp.zeros((16,), jnp.float32),
                    unroll=8)
                if t + _RING < _TPT:
                    _copy(t + _RING, k)
            tot = jnp.sum(acc) * (1.0 / _S)
            means_vec = jnp.where(lane == c_local, tot, means_vec)

        means_v[...] = means_vec
        pltpu.sync_copy(means_v, pub_hbm.at[cid, sid])
        plsc.subcore_barrier()

        @pl.when(sid == 0)
        def _finish():
            pltpu.sync_copy(nump_hbm.at[cid], nump_v)
            pltpu.sync_copy(denp_hbm.at[cid], denp_v)
            pltpu.sync_copy(pub_hbm.at[cid], stage_v)
            for i in range(4):
                sl = pl.ds(i * 16, 16)
                npos = nump_v[sl]
                dpos = denp_v[sl]
                nrow = lax.shift_right_logical(npos, 2)
                ncol = lax.bitwise_and(npos, 3)
                drow = lax.shift_right_logical(dpos, 2)
                dcol = lax.bitwise_and(dpos, 3)
                num = plsc.load_gather(stage_v, [nrow, ncol])
                den = plsc.load_gather(stage_v, [drow, dcol])
                ratio_v[sl] = num / den
            pltpu.sync_copy(ratio_v, out_hbm.at[pl.ds(cid * 64, 64)])

    return _s2_kernel


def kernel(x):
    x2 = x.reshape(_C, _S)
    ratios, _ = _make_s2_kernel()(x2,
                                  jnp.asarray(_CHAN),
                                  jnp.asarray(_NUMP),
                                  jnp.asarray(_DENP))
    scat_par = ratios[:_N_PAR]
    scat_per = ratios[28:28 + _N_PER]
    scat_quart = ratios[64:64 + _N_QUART]
    return (scat_par, scat_per, scat_quart)
